# baseline (speedup 1.0000x reference)
import functools
import jax
import jax.numpy as jnp
from jax import lax
from jax.experimental import pallas as pl
from jax.experimental.pallas import tpu as pltpu
from jax.experimental.pallas import tpu_sc as plsc

_SEQ, _BATCH, _D = 200, 4096, 32
_TOTAL = _SEQ * _BATCH
_V = 1_000_000
_NC, _NS, _L = 2, 16, 16
_NW = _NC * _NS
_SCALE = 8.0

_mesh = plsc.VectorSubcoreMesh(core_axis_name="c", subcore_axis_name="s")

_NWIN = 7813
_VPAD = _NWIN * 128
_W_BASE = _NWIN // _NW
_W_EXTRA = _NWIN % _NW


@functools.partial(
    pl.kernel,
    out_type=jax.ShapeDtypeStruct((_VPAD * _D,), jnp.float32),
    mesh=_mesh,
    scratch_types=[
        [pltpu.VMEM((_D, 128), jnp.float32) for _ in range(2)],
        [pltpu.VMEM((128 * _D,), jnp.float32) for _ in range(2)],
        [pltpu.SemaphoreType.DMA for _ in range(2)],
        [pltpu.SemaphoreType.DMA for _ in range(2)],
    ],
    compiler_params=pltpu.CompilerParams(
        use_tc_tiling_on_sc=True, needs_layout_passes=False
    ),
)
def _format_table(wt_hbm, tab_hbm, tbufs, stages, isems, osems):
    wid = lax.axis_index("s") * _NC + lax.axis_index("c")
    nwin = _W_BASE + jnp.where(wid < _W_EXTRA, 1, 0)
    win0 = wid * _W_BASE + jnp.minimum(wid, _W_EXTRA)

    def col0(i):
        return pl.multiple_of((win0 + i) * 128, 128)

    def tile_in(i, b):
        return pltpu.make_async_copy(
            wt_hbm.at[:, pl.ds(col0(i), 128)], tbufs[b], isems[b]
        )

    def row_out(i, b):
        return pltpu.make_async_copy(
            stages[b], tab_hbm.at[pl.ds(col0(i) * _D, 128 * _D)], osems[b]
        )

    iota = lax.iota(jnp.int32, _L)

    bases = [iota * _D + q * (_L * _D) for q in range(8)]

    def transpose_scale(b):
        @plsc.parallel_loop(0, _D, unroll=2)
        def _tr(d):
            dvec = jnp.broadcast_to(d, (_L,)).astype(jnp.int32)
            for q in range(8):
                v = tbufs[b][d, pl.ds(q * _L, _L)]
                plsc.store_scatter(stages[b], [bases[q] + dvec], v * _SCALE)

    for b in range(2):
        tile_in(b, b).start()

    @pl.loop(0, _W_BASE, step=2)
    def _win(g0):
        for b in range(2):
            g = g0 + b
            tile_in(g, b).wait()

            @pl.when(g0 >= 2)
            def _():
                row_out(g - 2, b).wait()

            transpose_scale(b)

            @pl.when(g + 2 < nwin)
            def _():
                tile_in(g + 2, b).start()

            row_out(g, b).start()

    for b in range(2):
        row_out(_W_BASE - 2 + b, b).wait()

    @pl.when(nwin > _W_BASE)
    def _tail():
        tile_in(_W_BASE, 0).wait()
        transpose_scale(0)
        row_out(_W_BASE, 0).start()
        row_out(_W_BASE, 0).wait()


_GPW = _TOTAL // 128 // _NW
_PER_W = _TOTAL // _NW


_ROW_BLK = _BATCH // 128 * 1024


@functools.partial(
    pl.kernel,
    out_type=jax.ShapeDtypeStruct((_SEQ, 4 * _ROW_BLK), jnp.float32),
    mesh=_mesh,
    scratch_types=[
        pltpu.VMEM((_PER_W,), jnp.int32),
        [pltpu.VMEM((128, _D), jnp.float32) for _ in range(2)],
        [pltpu.VMEM((4 * 8 * 128,), jnp.float32) for _ in range(2)],
        [pltpu.SemaphoreType.DMA for _ in range(2)],
        [pltpu.SemaphoreType.DMA for _ in range(2)],
    ],
    compiler_params=pltpu.CompilerParams(
        use_tc_tiling_on_sc=False, needs_layout_passes=False
    ),
)
def _lookup(idx_hbm, tab_hbm, out_hbm, idx_v, rows, obufs, gsems, osems):
    wid = lax.axis_index("s") * _NC + lax.axis_index("c")
    base = wid * _PER_W
    g0 = wid * _GPW

    pltpu.sync_copy(idx_hbm.at[pl.ds(base, _PER_W)], idx_v)

    def gather(g, b):
        src = tab_hbm.at[idx_v.at[pl.ds(g * 128, 128)]]
        return pltpu.make_async_copy(src, rows[b], gsems[b])

    def writeback(g, b):
        gg = g0 + g
        s = gg // (_BATCH // 128)
        boff = (gg % (_BATCH // 128)) * 1024
        return [
            pltpu.make_async_copy(
                obufs[b].at[pl.ds(dt * 1024, 1024)],
                out_hbm.at[s, pl.ds(dt * _ROW_BLK + boff, 1024)],
                osems[b],
            )
            for dt in range(4)
        ]

    iota = lax.iota(jnp.int32, _L)
    obases = [iota * 128 + h * (_L * 128) for h in range(2)]

    for b in range(2):
        gather(b, b).start()

    @pl.loop(0, _GPW, step=2)
    def _grp(gg0):
        for b in range(2):
            g = gg0 + b
            gather(g, b).wait()

            @pl.when(gg0 >= 2)
            def _():
                for c in writeback(g - 2, b):
                    c.wait()

            @plsc.parallel_loop(0, 128, unroll=4)
            def _tr(bi):
                bvec = jnp.broadcast_to(bi, (_L,)).astype(jnp.int32)
                for h in range(2):
                    v = rows[b][bi, pl.ds(h * _L, _L)]
                    plsc.store_scatter(obufs[b], [obases[h] + bvec], v)

            @pl.when(gg0 < _GPW - 2)
            def _():
                gather(g + 2, b).start()

            for c in writeback(g, b):
                c.start()

    for b in range(2):
        for c in writeback(_GPW - 2 + b, b):
            c.wait()


def kernel(src, weight):
    tab = _format_table(weight.T)
    flat = src.reshape(_TOTAL)
    out2 = _lookup(flat, tab.reshape(_VPAD, _D))
    out5 = out2.reshape(_SEQ, 4, _BATCH // 128, 8, 128)
    return out5.transpose(0, 2, 4, 1, 3).reshape(_SEQ, _BATCH, _D)
